# baseline (speedup 1.0000x reference)
import functools

import jax
import jax.numpy as jnp
from jax import lax
from jax.experimental import pallas as pl
from jax.experimental.pallas import tpu as pltpu
from jax.experimental.pallas import tpu_sc as plsc

K_NN = 12
S_PTS = 32
_EPS = 1e-5
_BIG = 3.4e38

_SC_CORES = 2
_SC_SUBCORES = 16
_SC_WORKERS = _SC_CORES * _SC_SUBCORES


def _fps_knn_body(n, bsz, xyzt_ref, xyzs_ref, idx_ref):
    ns = n // 128
    q_tot = 2 * S_PTS * bsz

    xt, yt, zt = xyzt_ref[0], xyzt_ref[1], xyzt_ref[2]
    xs, ys, zs = xyzs_ref[0], xyzs_ref[1], xyzs_ref[2]

    si = lax.broadcasted_iota(jnp.int32, (bsz, ns, 128), 1)
    li = lax.broadcasted_iota(jnp.int32, (bsz, ns, 128), 2)
    fi = si * 128 + li

    def red2(a, op):
        return op(op(a, axis=2, keepdims=True), axis=1, keepdims=True)

    dist = jnp.full((bsz, ns, 128), 1e10, dtype=jnp.float32)
    far = jnp.zeros((bsz, 1, 1), dtype=jnp.int32)
    centroids = []
    for _ in range(S_PTS):
        oh = fi == far
        cx = red2(jnp.where(oh, xt, 0.0), jnp.sum)
        cy = red2(jnp.where(oh, yt, 0.0), jnp.sum)
        cz = red2(jnp.where(oh, zt, 0.0), jnp.sum)
        centroids.append((cx, cy, cz))
        dx, dy, dz = xt - cx, yt - cy, zt - cz
        d = dx * dx + dy * dy + dz * dz
        dist = jnp.minimum(dist, d)
        m = red2(dist, jnp.max)
        far = red2(jnp.where(dist == m, fi, n), jnp.min)

    rows = []
    for cx, cy, cz in centroids:
        dx, dy, dz = xt - cx, yt - cy, zt - cz
        rows.append(dx * dx + dy * dy + dz * dz)
    for cx, cy, cz in centroids:
        dx, dy, dz = xs - cx, ys - cy, zs - cz
        rows.append(dx * dx + dy * dy + dz * dz)
    d3 = jnp.concatenate(rows, axis=0)

    li2 = lax.broadcasted_iota(jnp.int32, (q_tot, 128), 1)
    si2 = lax.broadcasted_iota(jnp.int32, (q_tot, ns), 1)
    li3 = lax.broadcasted_iota(jnp.int32, (q_tot, ns, 128), 2)

    cm = jnp.min(d3, axis=1)
    cnt = jnp.zeros((q_tot, 128), dtype=jnp.int32)
    cols = []
    for _ in range(K_NN):
        m = jnp.min(cm, axis=1, keepdims=True)
        lane = jnp.min(jnp.where(cm == m, li2, 128), axis=1, keepdims=True)
        col = jnp.min(jnp.where(li3 == lane[:, :, None], d3, _BIG), axis=2)
        csel = jnp.sum(jnp.where(li2 == lane, cnt, 0), axis=1, keepdims=True)
        less = jnp.sum((col < m).astype(jnp.int32), axis=1, keepdims=True)
        e1 = csel - less + 1
        elig = col == m
        cs = elig.astype(jnp.int32)
        sh = 1
        while sh < ns:
            cs = cs + jnp.concatenate(
                [jnp.zeros((q_tot, sh), jnp.int32), cs[:, : ns - sh]], axis=1)
            sh *= 2
        sstar = jnp.min(jnp.where(elig & (cs == e1), si2, ns),
                        axis=1, keepdims=True)
        cols.append(sstar * 128 + lane)
        total = jnp.sum(elig.astype(jnp.int32), axis=1, keepdims=True)
        m1 = jnp.min(jnp.where(col > m, col, _BIG), axis=1, keepdims=True)
        newmin = jnp.where(total - e1 > 0, m, m1)
        cm = jnp.where(li2 == lane, newmin, cm)
        cnt = jnp.where(li2 == lane, cnt + 1, cnt)

    idx = jnp.concatenate(cols, axis=1)
    ri = lax.broadcasted_iota(jnp.int32, (q_tot, K_NN), 0)
    idx_ref[...] = idx + (ri % bsz) * n


def _fps_knn(xyzt4, xyzs4):
    _, bsz, ns, _ = xyzt4.shape
    n = ns * 128
    q_tot = 2 * S_PTS * bsz
    return pl.pallas_call(
        functools.partial(_fps_knn_body, n, bsz),
        out_shape=jax.ShapeDtypeStruct((q_tot, K_NN), jnp.int32),
    )(xyzt4, xyzs4)


def _gather_rows(tbl_s, flat_s, tbl_t, flat_t):
    n_rows = flat_s.shape[0]
    per = n_rows // _SC_WORKERS
    cs = tbl_s.shape[1]
    ct = tbl_t.shape[1]
    mesh = plsc.VectorSubcoreMesh(
        core_axis_name="c",
        subcore_axis_name="s",
        num_cores=_SC_CORES,
        num_subcores=_SC_SUBCORES,
    )

    @functools.partial(
        pl.kernel,
        mesh=mesh,
        out_type=[
            jax.ShapeDtypeStruct((n_rows, cs), jnp.float32),
            jax.ShapeDtypeStruct((n_rows, ct), jnp.float32),
        ],
        scratch_types=[
            pltpu.VMEM((per,), jnp.int32),
            pltpu.VMEM((per, cs), jnp.float32),
            pltpu.VMEM((per,), jnp.int32),
            pltpu.VMEM((per, ct), jnp.float32),
            pltpu.SemaphoreType.DMA,
            pltpu.SemaphoreType.DMA,
        ],
    )
    def gather_k(tbls, idxs, tblt, idxt, out_s, out_t,
                 idxv_s, rows_s, idxv_t, rows_t, sem_s, sem_t):
        wid = lax.axis_index("s") * _SC_CORES + lax.axis_index("c")
        base = wid * per
        pltpu.sync_copy(idxs.at[pl.ds(base, per)], idxv_s)
        pltpu.sync_copy(idxt.at[pl.ds(base, per)], idxv_t)
        cp_s = pltpu.async_copy(tbls.at[idxv_s], rows_s, sem_s)
        cp_t = pltpu.async_copy(tblt.at[idxv_t], rows_t, sem_t)
        cp_s.wait()
        pltpu.sync_copy(rows_s, out_s.at[pl.ds(base, per)])
        cp_t.wait()
        pltpu.sync_copy(rows_t, out_t.at[pl.ds(base, per)])

    return gather_k(tbl_s, flat_s, tbl_t, flat_t)


def _dense_body(gs_ref, gt_ref, ws_ref, wt_ref, ps_ref, pt_ref,
                outs_ref, outt_ref):
    def branch(g, w, p, out_ref):
        y = lax.dot_general(g.astype(jnp.bfloat16), w.astype(jnp.bfloat16),
                            (((1,), (1,)), ((), ())),
                            preferred_element_type=jnp.float32)
        y = y + p[0:1, :]
        mean = jnp.mean(y, axis=0, keepdims=True)
        c = y - mean
        var = jnp.mean(c * c, axis=0, keepdims=True)
        z = p[1:2, :] * (c / jnp.sqrt(var + _EPS)) + p[2:3, :]
        z = jnp.maximum(z, 0.0)
        nrow = out_ref.shape[0]
        acc = z[0:nrow, :]
        for k in range(1, K_NN):
            acc = jnp.maximum(acc, z[k * nrow : (k + 1) * nrow, :])
        out_ref[...] = acc

    branch(gs_ref[...], ws_ref[...], ps_ref[...], outs_ref)
    branch(gt_ref[...], wt_ref[...], pt_ref[...], outt_ref)


def _dense(g_s, g_t, Ws, Wt, ps, pt, n_groups):
    n_rows, cs = g_s.shape
    ct = g_t.shape[1]
    o = Ws.shape[0]
    ch = 256
    grid = o // ch
    return pl.pallas_call(
        _dense_body,
        grid=(grid,),
        in_specs=[
            pl.BlockSpec((n_rows, cs), lambda j: (0, 0)),
            pl.BlockSpec((n_rows, ct), lambda j: (0, 0)),
            pl.BlockSpec((ch, cs), lambda j: (j, 0)),
            pl.BlockSpec((ch, ct), lambda j: (j, 0)),
            pl.BlockSpec((3, ch), lambda j: (0, j)),
            pl.BlockSpec((3, ch), lambda j: (0, j)),
        ],
        out_specs=[
            pl.BlockSpec((n_groups, ch), lambda j: (0, j)),
            pl.BlockSpec((n_groups, ch), lambda j: (0, j)),
        ],
        out_shape=[
            jax.ShapeDtypeStruct((n_groups, o), jnp.float32),
            jax.ShapeDtypeStruct((n_groups, o), jnp.float32),
        ],
    )(g_s, g_t, Ws, Wt, ps, pt)


def kernel(feature_s, xyz_s, feature_t, xyz_t,
           Ws, bs, gamma_s, beta_s, Wt, bt, gamma_t, beta_t):
    bsz, n, cs = feature_s.shape
    ct = feature_t.shape[2]
    o = Ws.shape[0]

    xyzt4 = jnp.transpose(xyz_t, (2, 0, 1)).reshape(3, bsz, n // 128, 128)
    xyzs4 = jnp.transpose(xyz_s, (2, 0, 1)).reshape(3, bsz, n // 128, 128)
    idx = _fps_knn(xyzt4, xyzs4)

    idxc = idx.reshape(2, S_PTS, bsz, K_NN)
    perm = jnp.transpose(idxc, (0, 3, 2, 1))
    flat_t = perm[0].reshape(-1)
    flat_s = perm[1].reshape(-1)

    g_s, g_t = _gather_rows(
        feature_s.reshape(bsz * n, cs), flat_s,
        feature_t.reshape(bsz * n, ct), flat_t,
    )

    ps = jnp.stack([bs, gamma_s, beta_s])
    pt = jnp.stack([bt, gamma_t, beta_t])
    n_groups = bsz * S_PTS
    out_s, out_t = _dense(g_s, g_t, Ws, Wt, ps, pt, n_groups)
    return (out_s.reshape(bsz, S_PTS, o), out_t.reshape(bsz, S_PTS, o))


# speedup vs baseline: 4.6863x; 4.6863x over previous
import functools

import jax
import jax.numpy as jnp
from jax import lax
from jax.experimental import pallas as pl
from jax.experimental.pallas import tpu as pltpu
from jax.experimental.pallas import tpu_sc as plsc

K_NN = 12
S_PTS = 32
_EPS = 1e-5
_BIG = 3.4e38

_SC_CORES = 2
_SC_SUBCORES = 16
_SC_WORKERS = _SC_CORES * _SC_SUBCORES


def _fps_knn_body(n, bsz, xyzt_ref, xyzs_ref, idx_ref, d3_ref):
    ns = n // 128
    q_cloud = S_PTS * bsz
    q_tot = 2 * q_cloud

    xt, yt, zt = xyzt_ref[0], xyzt_ref[1], xyzt_ref[2]
    xs, ys, zs = xyzs_ref[0], xyzs_ref[1], xyzs_ref[2]

    si = lax.broadcasted_iota(jnp.int32, (bsz, ns, 128), 1)
    li = lax.broadcasted_iota(jnp.int32, (bsz, ns, 128), 2)
    fi = si * 128 + li

    def red2(a, op):
        return op(op(a, axis=2, keepdims=True), axis=1, keepdims=True)

    dist = jnp.full((bsz, ns, 128), 1e10, dtype=jnp.float32)
    far = jnp.zeros((bsz, 1, 1), dtype=jnp.int32)
    centroids = []
    for _ in range(S_PTS):
        oh = fi == far
        cx = red2(jnp.where(oh, xt, 0.0), jnp.sum)
        cy = red2(jnp.where(oh, yt, 0.0), jnp.sum)
        cz = red2(jnp.where(oh, zt, 0.0), jnp.sum)
        centroids.append((cx, cy, cz))
        dx, dy, dz = xt - cx, yt - cy, zt - cz
        d = dx * dx + dy * dy + dz * dz
        dist = jnp.minimum(dist, d)
        m = red2(dist, jnp.max)
        far = red2(jnp.where(dist == m, fi, n), jnp.min)

    for qi, (cx, cy, cz) in enumerate(centroids):
        dx, dy, dz = xt - cx, yt - cy, zt - cz
        d3_ref[0, qi * bsz : (qi + 1) * bsz] = dx * dx + dy * dy + dz * dz
    for qi, (cx, cy, cz) in enumerate(centroids):
        dx, dy, dz = xs - cx, ys - cy, zs - cz
        d3_ref[1, qi * bsz : (qi + 1) * bsz] = dx * dx + dy * dy + dz * dz

    li2 = lax.broadcasted_iota(jnp.int32, (q_tot, 128), 1)
    si2 = lax.broadcasted_iota(jnp.int32, (q_tot, ns), 1)
    li4 = lax.broadcasted_iota(jnp.int32, (q_cloud, ns, 128), 2)

    cm = jnp.concatenate(
        [jnp.min(d3_ref[0], axis=1), jnp.min(d3_ref[1], axis=1)], axis=0)
    cnt = jnp.zeros((q_tot, 128), dtype=jnp.int32)
    cols = []
    for _ in range(K_NN):
        m = jnp.min(cm, axis=1, keepdims=True)
        lane = jnp.min(jnp.where(cm == m, li2, 128), axis=1, keepdims=True)
        col = jnp.concatenate([
            jnp.min(jnp.where(
                li4 == lane[c * q_cloud : (c + 1) * q_cloud][:, :, None],
                d3_ref[c], _BIG), axis=2)
            for c in range(2)], axis=0)
        csel = jnp.sum(jnp.where(li2 == lane, cnt, 0), axis=1, keepdims=True)
        less = jnp.sum((col < m).astype(jnp.int32), axis=1, keepdims=True)
        e1 = csel - less + 1
        elig = col == m
        cs = elig.astype(jnp.int32)
        sh = 1
        while sh < ns:
            cs = cs + jnp.concatenate(
                [jnp.zeros((q_tot, sh), jnp.int32), cs[:, : ns - sh]], axis=1)
            sh *= 2
        sstar = jnp.min(jnp.where(elig & (cs == e1), si2, ns),
                        axis=1, keepdims=True)
        cols.append(sstar * 128 + lane)
        total = jnp.sum(elig.astype(jnp.int32), axis=1, keepdims=True)
        m1 = jnp.min(jnp.where(col > m, col, _BIG), axis=1, keepdims=True)
        newmin = jnp.where(total - e1 > 0, m, m1)
        cm = jnp.where(li2 == lane, newmin, cm)
        cnt = jnp.where(li2 == lane, cnt + 1, cnt)

    idx = jnp.concatenate(cols, axis=1)
    ri = lax.broadcasted_iota(jnp.int32, (q_tot, K_NN), 0)
    idx_ref[...] = idx + (ri % bsz) * n


def _fps_knn(xyzt4, xyzs4):
    _, bsz, ns, _ = xyzt4.shape
    n = ns * 128
    q_tot = 2 * S_PTS * bsz
    return pl.pallas_call(
        functools.partial(_fps_knn_body, n, bsz),
        out_shape=jax.ShapeDtypeStruct((q_tot, K_NN), jnp.int32),
        scratch_shapes=[pltpu.VMEM((2, S_PTS * bsz, ns, 128), jnp.float32)],
    )(xyzt4, xyzs4)


def _gather_rows(tbl_s, flat_s, tbl_t, flat_t):
    n_rows = flat_s.shape[0]
    per = n_rows // _SC_WORKERS
    cs = tbl_s.shape[1]
    ct = tbl_t.shape[1]
    mesh = plsc.VectorSubcoreMesh(
        core_axis_name="c",
        subcore_axis_name="s",
        num_cores=_SC_CORES,
        num_subcores=_SC_SUBCORES,
    )

    @functools.partial(
        pl.kernel,
        mesh=mesh,
        out_type=[
            jax.ShapeDtypeStruct((n_rows, cs), jnp.float32),
            jax.ShapeDtypeStruct((n_rows, ct), jnp.float32),
        ],
        scratch_types=[
            pltpu.VMEM((per,), jnp.int32),
            pltpu.VMEM((per, cs), jnp.float32),
            pltpu.VMEM((per,), jnp.int32),
            pltpu.VMEM((per, ct), jnp.float32),
            pltpu.SemaphoreType.DMA,
            pltpu.SemaphoreType.DMA,
        ],
    )
    def gather_k(tbls, idxs, tblt, idxt, out_s, out_t,
                 idxv_s, rows_s, idxv_t, rows_t, sem_s, sem_t):
        wid = lax.axis_index("s") * _SC_CORES + lax.axis_index("c")
        base = wid * per
        pltpu.sync_copy(idxs.at[pl.ds(base, per)], idxv_s)
        pltpu.sync_copy(idxt.at[pl.ds(base, per)], idxv_t)
        cp_s = pltpu.async_copy(tbls.at[idxv_s], rows_s, sem_s)
        cp_t = pltpu.async_copy(tblt.at[idxv_t], rows_t, sem_t)
        cp_s.wait()
        pltpu.sync_copy(rows_s, out_s.at[pl.ds(base, per)])
        cp_t.wait()
        pltpu.sync_copy(rows_t, out_t.at[pl.ds(base, per)])

    return gather_k(tbl_s, flat_s, tbl_t, flat_t)


def _dense_body(gs_ref, gt_ref, ws_ref, wt_ref, ps_ref, pt_ref,
                outs_ref, outt_ref):
    def branch(g, w, p, out_ref):
        y = lax.dot_general(g.astype(jnp.bfloat16), w.astype(jnp.bfloat16),
                            (((1,), (1,)), ((), ())),
                            preferred_element_type=jnp.float32)
        y = y + p[0:1, :]
        mean = jnp.mean(y, axis=0, keepdims=True)
        c = y - mean
        var = jnp.mean(c * c, axis=0, keepdims=True)
        z = p[1:2, :] * (c / jnp.sqrt(var + _EPS)) + p[2:3, :]
        z = jnp.maximum(z, 0.0)
        nrow = out_ref.shape[0]
        acc = z[0:nrow, :]
        for k in range(1, K_NN):
            acc = jnp.maximum(acc, z[k * nrow : (k + 1) * nrow, :])
        out_ref[...] = acc

    branch(gs_ref[...], ws_ref[...], ps_ref[...], outs_ref)
    branch(gt_ref[...], wt_ref[...], pt_ref[...], outt_ref)


def _dense(g_s, g_t, Ws, Wt, ps, pt, n_groups):
    n_rows, cs = g_s.shape
    ct = g_t.shape[1]
    o = Ws.shape[0]
    ch = 256
    grid = o // ch
    return pl.pallas_call(
        _dense_body,
        grid=(grid,),
        in_specs=[
            pl.BlockSpec((n_rows, cs), lambda j: (0, 0)),
            pl.BlockSpec((n_rows, ct), lambda j: (0, 0)),
            pl.BlockSpec((ch, cs), lambda j: (j, 0)),
            pl.BlockSpec((ch, ct), lambda j: (j, 0)),
            pl.BlockSpec((3, ch), lambda j: (0, j)),
            pl.BlockSpec((3, ch), lambda j: (0, j)),
        ],
        out_specs=[
            pl.BlockSpec((n_groups, ch), lambda j: (0, j)),
            pl.BlockSpec((n_groups, ch), lambda j: (0, j)),
        ],
        out_shape=[
            jax.ShapeDtypeStruct((n_groups, o), jnp.float32),
            jax.ShapeDtypeStruct((n_groups, o), jnp.float32),
        ],
    )(g_s, g_t, Ws, Wt, ps, pt)


def kernel(feature_s, xyz_s, feature_t, xyz_t,
           Ws, bs, gamma_s, beta_s, Wt, bt, gamma_t, beta_t):
    bsz, n, cs = feature_s.shape
    ct = feature_t.shape[2]
    o = Ws.shape[0]

    xyzt4 = jnp.transpose(xyz_t, (2, 0, 1)).reshape(3, bsz, n // 128, 128)
    xyzs4 = jnp.transpose(xyz_s, (2, 0, 1)).reshape(3, bsz, n // 128, 128)
    idx = _fps_knn(xyzt4, xyzs4)

    idxc = idx.reshape(2, S_PTS, bsz, K_NN)
    perm = jnp.transpose(idxc, (0, 3, 2, 1))
    flat_t = perm[0].reshape(-1)
    flat_s = perm[1].reshape(-1)

    g_s, g_t = _gather_rows(
        feature_s.reshape(bsz * n, cs), flat_s,
        feature_t.reshape(bsz * n, ct), flat_t,
    )

    ps = jnp.stack([bs, gamma_s, beta_s])
    pt = jnp.stack([bt, gamma_t, beta_t])
    n_groups = bsz * S_PTS
    out_s, out_t = _dense(g_s, g_t, Ws, Wt, ps, pt, n_groups)
    return (out_s.reshape(bsz, S_PTS, o), out_t.reshape(bsz, S_PTS, o))
